# Initial kernel scaffold; baseline (speedup 1.0000x reference)
import jax
import jax.numpy as jnp
from jax.experimental import pallas as pl


def kernel(word_vector, weight):
    raise NotImplementedError("write your pallas kernel here")


# speedup vs baseline: 2.9679x; 2.9679x over previous
import functools

import jax
import jax.numpy as jnp
from jax import lax
from jax.experimental import pallas as pl
from jax.experimental.pallas import tpu as pltpu
from jax.experimental.pallas import tpu_sc as plsc

VOCAB = 100000
DIM = 128
BATCH = 4096
HIST = 50

_NC = 2
_NS = 16
_NW = _NC * _NS

_TOTAL = BATCH * HIST
_PER_W = _TOTAL // _NW
_CHUNK = 128
_NCHUNK = _PER_W // _CHUNK


def _embed_grid(idx_hbm, table_hbm, out_hbm, idx_v, rows_v, sem):
    w = lax.axis_index("s") * _NC + lax.axis_index("c")
    base = w * _PER_W
    pltpu.sync_copy(idx_hbm.at[w], idx_v)

    def chunk(j, carry):
        pltpu.async_copy(table_hbm.at[idx_v.at[j]], rows_v, sem).wait()
        pltpu.sync_copy(rows_v, out_hbm.at[pl.ds(base + j * _CHUNK, _CHUNK)])
        return carry

    lax.fori_loop(0, _NCHUNK, chunk, 0)


@jax.jit
def _embed(idx, table):
    mesh = plsc.VectorSubcoreMesh(core_axis_name="c", subcore_axis_name="s")
    k = functools.partial(
        pl.kernel,
        out_type=jax.ShapeDtypeStruct((_TOTAL, DIM), jnp.float32),
        mesh=mesh,
        scratch_types=[
            pltpu.VMEM((_NCHUNK, _CHUNK), jnp.int32),
            pltpu.VMEM((_CHUNK, DIM), jnp.float32),
            pltpu.SemaphoreType.DMA,
        ],
    )(_embed_grid)
    return k(idx, table)


def kernel(word_vector, weight):
    idx = word_vector.reshape(_NW, _NCHUNK, _CHUNK).astype(jnp.int32)
    out = _embed(idx, weight)
    return out.reshape(BATCH, HIST, DIM)


# speedup vs baseline: 3.3595x; 1.1319x over previous
import functools

import jax
import jax.numpy as jnp
from jax import lax
from jax.experimental import pallas as pl
from jax.experimental.pallas import tpu as pltpu
from jax.experimental.pallas import tpu_sc as plsc

VOCAB = 100000
DIM = 128
BATCH = 4096
HIST = 50

_NC = 2
_NS = 16
_NW = _NC * _NS

_TOTAL = BATCH * HIST
_PER_W = _TOTAL // _NW
_CHUNK = 128
_NCHUNK = _PER_W // _CHUNK


_NBUF = 5
_OUTER = _NCHUNK // _NBUF - 1


def _embed_grid(idx_hbm, table_hbm, out_hbm, idx_v, *bufs):
    rows = bufs[:_NBUF]
    gsem = bufs[_NBUF:2 * _NBUF]
    ssem = bufs[2 * _NBUF:]
    w = lax.axis_index("s") * _NC + lax.axis_index("c")
    base = w * _PER_W
    pltpu.sync_copy(idx_hbm.at[w], idx_v)

    def start_gather(b, j):
        pltpu.async_copy(table_hbm.at[idx_v.at[j]], rows[b], gsem[b])

    def wait_gather(b):
        pltpu.make_async_copy(table_hbm.at[pl.ds(0, _CHUNK)], rows[b],
                              gsem[b]).wait()

    def start_store(b, j):
        pltpu.async_copy(rows[b], out_hbm.at[pl.ds(base + j * _CHUNK, _CHUNK)],
                         ssem[b])

    def wait_store(b):
        pltpu.make_async_copy(rows[b], out_hbm.at[pl.ds(base, _CHUNK)],
                              ssem[b]).wait()

    for b in range(_NBUF):
        start_gather(b, b)

    def outer(t, carry):
        for b in range(_NBUF):
            j = t * _NBUF + b
            wait_gather(b)
            start_store(b, j)
            wait_store(b)
            start_gather(b, j + _NBUF)
        return carry

    lax.fori_loop(0, _OUTER, outer, 0)
    for b in range(_NBUF):
        wait_gather(b)
        start_store(b, _OUTER * _NBUF + b)
    for b in range(_NBUF):
        wait_store(b)


@jax.jit
def _embed(idx, table):
    mesh = plsc.VectorSubcoreMesh(core_axis_name="c", subcore_axis_name="s")
    k = functools.partial(
        pl.kernel,
        out_type=jax.ShapeDtypeStruct((_TOTAL, DIM), jnp.float32),
        mesh=mesh,
        scratch_types=(
            [pltpu.VMEM((_NCHUNK, _CHUNK), jnp.int32)]
            + [pltpu.VMEM((_CHUNK, DIM), jnp.float32) for _ in range(_NBUF)]
            + [pltpu.SemaphoreType.DMA for _ in range(2 * _NBUF)]
        ),
    )(_embed_grid)
    return k(idx, table)


def kernel(word_vector, weight):
    idx = word_vector.reshape(_NW, _NCHUNK, _CHUNK).astype(jnp.int32)
    out = _embed(idx, weight)
    return out.reshape(BATCH, HIST, DIM)


# speedup vs baseline: 5.8578x; 1.7437x over previous
import functools

import jax
import jax.numpy as jnp
from jax import lax
from jax.experimental import pallas as pl
from jax.experimental.pallas import tpu as pltpu
from jax.experimental.pallas import tpu_sc as plsc

VOCAB = 100000
DIM = 128
BATCH = 4096
HIST = 50

_NC = 2
_NS = 16
_NW = _NC * _NS

_B_PER_W = BATCH // _NW
_FILL = 8
_NFILL = _B_PER_W // _FILL
_NBUF = 2


def _embed_grid(idx_hbm, table_hbm, out_hbm, idx_v, *bufs):
    rows = bufs[:_NBUF]
    gsem = bufs[_NBUF:2 * _NBUF]
    ssem = bufs[2 * _NBUF:]
    w = lax.axis_index("s") * _NC + lax.axis_index("c")
    bbase = w * _B_PER_W
    pltpu.sync_copy(idx_hbm.at[pl.ds(bbase, _B_PER_W)], idx_v)

    def start_fill(b, f):
        for i in range(_FILL):
            pltpu.async_copy(table_hbm.at[idx_v.at[_FILL * f + i]],
                             rows[b].at[i], gsem[b])

    def wait_fill(b):
        pltpu.make_async_copy(out_hbm.at[pl.ds(0, _FILL)], rows[b],
                              gsem[b]).wait()

    def start_store(b, f):
        pltpu.async_copy(rows[b], out_hbm.at[pl.ds(bbase + _FILL * f, _FILL)],
                         ssem[b])

    def wait_store(b):
        pltpu.make_async_copy(rows[b], out_hbm.at[pl.ds(0, _FILL)],
                              ssem[b]).wait()

    for b in range(_NBUF):
        start_fill(b, b)

    def outer(t, carry):
        for b in range(_NBUF):
            f = t * _NBUF + b
            wait_fill(b)
            start_store(b, f)
            wait_store(b)
            start_fill(b, f + _NBUF)
        return carry

    lax.fori_loop(0, _NFILL // _NBUF - 1, outer, 0)
    for b in range(_NBUF):
        f = _NFILL - _NBUF + b
        wait_fill(b)
        start_store(b, f)
    for b in range(_NBUF):
        wait_store(b)


@jax.jit
def _embed(idx, table):
    mesh = plsc.VectorSubcoreMesh(core_axis_name="c", subcore_axis_name="s")
    k = functools.partial(
        pl.kernel,
        out_type=jax.ShapeDtypeStruct((BATCH, HIST, DIM), jnp.float32),
        mesh=mesh,
        scratch_types=(
            [pltpu.VMEM((_B_PER_W, HIST), jnp.int32)]
            + [pltpu.VMEM((_FILL, HIST, DIM), jnp.float32)
               for _ in range(_NBUF)]
            + [pltpu.SemaphoreType.DMA for _ in range(2 * _NBUF)]
        ),
    )(_embed_grid)
    return k(idx, table)


def kernel(word_vector, weight):
    return _embed(word_vector.astype(jnp.int32), weight)


# speedup vs baseline: 5.9417x; 1.0143x over previous
import functools

import jax
import jax.numpy as jnp
from jax import lax
from jax.experimental import pallas as pl
from jax.experimental.pallas import tpu as pltpu
from jax.experimental.pallas import tpu_sc as plsc

VOCAB = 100000
DIM = 128
BATCH = 4096
HIST = 50

_NC = 2
_NS = 16
_NW = _NC * _NS

_B_PER_W = BATCH // _NW
_FILL = 8
_NFILL = _B_PER_W // _FILL
_NBUF = 2


def _embed_grid(idx_hbm, table_hbm, out_hbm, idx_v, *bufs):
    rows = bufs[:_NBUF]
    gsem = bufs[_NBUF:2 * _NBUF]
    ssem = bufs[2 * _NBUF:]
    w = lax.axis_index("s") * _NC + lax.axis_index("c")
    bbase = w * _B_PER_W
    pltpu.sync_copy(idx_hbm.at[pl.ds(bbase, _B_PER_W)], idx_v)

    def start_fill(b, f):
        for i in range(_FILL):
            pltpu.async_copy(table_hbm.at[idx_v.at[_FILL * f + i]],
                             rows[b].at[i], gsem[b])

    def wait_fill(b):
        pltpu.make_async_copy(out_hbm.at[pl.ds(0, _FILL)], rows[b],
                              gsem[b]).wait()

    def start_store(b, f):
        pltpu.async_copy(rows[b], out_hbm.at[pl.ds(bbase + _FILL * f, _FILL)],
                         ssem[b])

    def wait_store(b):
        pltpu.make_async_copy(rows[b], out_hbm.at[pl.ds(0, _FILL)],
                              ssem[b]).wait()

    for b in range(_NBUF):
        start_fill(b, b)

    def outer(t, carry):
        for b in range(_NBUF):
            f = t * _NBUF + b
            wait_fill(b)
            start_store(b, f)
            wait_store(b)
            start_fill(b, f + _NBUF)
        return carry

    lax.fori_loop(0, _NFILL // _NBUF - 1, outer, 0)
    for b in range(_NBUF):
        f = _NFILL - _NBUF + b
        wait_fill(b)
        start_store(b, f)
    for b in range(_NBUF):
        wait_store(b)


@jax.jit
def _embed(idx, table):
    mesh = plsc.VectorSubcoreMesh(core_axis_name="c", subcore_axis_name="s")
    k = functools.partial(
        pl.kernel,
        out_type=jax.ShapeDtypeStruct((BATCH, HIST, DIM), jnp.float32),
        mesh=mesh,
        compiler_params=pltpu.CompilerParams(use_tc_tiling_on_sc=True),
        scratch_types=(
            [pltpu.VMEM((_B_PER_W, HIST), jnp.int32)]
            + [pltpu.VMEM((_FILL, HIST, DIM), jnp.float32)
               for _ in range(_NBUF)]
            + [pltpu.SemaphoreType.DMA for _ in range(2 * _NBUF)]
        ),
    )(_embed_grid)
    return k(idx, table)


def kernel(word_vector, weight):
    return _embed(word_vector.astype(jnp.int32), weight)
